# baseline (speedup 1.0000x reference)
import functools

import jax
import jax.numpy as jnp
from jax.experimental import pallas as pl
from jax.experimental.pallas import tpu as pltpu

N = 100000
B = 512
_BN = 1024


def _first_layer_body(x_ref, deg_ref, w_ref, y_ref, dis_ref):
    dis = jax.lax.rsqrt(deg_ref[...] + 1.0)
    dis_ref[...] = dis
    y_ref[...] = dis * (x_ref[...] @ w_ref[...])


def _mid_layer_body(agg_ref, y_ref, dis_ref, b_ref, w_ref, out_ref):
    dis = dis_ref[...]
    h = jnp.maximum(dis * (agg_ref[...] + y_ref[...]) + b_ref[...], 0.0)
    out_ref[...] = dis * (h @ w_ref[...])


def _pool_body(agg_ref, y_ref, dis_ref, b_ref, batch_ref, sums_ref, cnt_ref):
    i = pl.program_id(0)

    @pl.when(i == 0)
    def _():
        sums_ref[...] = jnp.zeros_like(sums_ref)
        cnt_ref[...] = jnp.zeros_like(cnt_ref)

    dis = dis_ref[...]
    h = jnp.maximum(dis * (agg_ref[...] + y_ref[...]) + b_ref[...], 0.0)
    rows = i * _BN + jax.lax.broadcasted_iota(jnp.int32, (_BN, 1), 0)
    valid = rows < N
    seg = jax.lax.broadcasted_iota(jnp.int32, (_BN, B), 1)
    onehot = jnp.where((batch_ref[...] == seg) & valid, 1.0, 0.0)
    sums_ref[...] += jax.lax.dot_general(
        onehot, h, (((0,), (0,)), ((), ())), preferred_element_type=jnp.float32)
    cnt_ref[...] += jnp.sum(onehot, axis=0, keepdims=True)


def _final_body(sums_ref, cnt_ref, w_ref, b_ref, o_ref):
    pooled = sums_ref[...] / jnp.maximum(cnt_ref[...].T, 1.0)
    o_ref[...] = pooled @ w_ref[...] + b_ref[...]


def _row_grid(d_in, d_out, extra_outs=()):
    grid = (pl.cdiv(N, _BN),)
    return dict(grid=grid)


def _tc_first_layer(x, deg, W):
    grid = (pl.cdiv(N, _BN),)
    return pl.pallas_call(
        _first_layer_body,
        grid=grid,
        in_specs=[
            pl.BlockSpec((_BN, x.shape[1]), lambda i: (i, 0)),
            pl.BlockSpec((_BN, 1), lambda i: (i, 0)),
            pl.BlockSpec(W.shape, lambda i: (0, 0)),
        ],
        out_specs=[
            pl.BlockSpec((_BN, W.shape[1]), lambda i: (i, 0)),
            pl.BlockSpec((_BN, 1), lambda i: (i, 0)),
        ],
        out_shape=[
            jax.ShapeDtypeStruct((N, W.shape[1]), jnp.float32),
            jax.ShapeDtypeStruct((N, 1), jnp.float32),
        ],
    )(x, deg, W)


def _tc_mid_layer(agg, y, dis, b, W):
    grid = (pl.cdiv(N, _BN),)
    d_in = y.shape[1]
    return pl.pallas_call(
        _mid_layer_body,
        grid=grid,
        in_specs=[
            pl.BlockSpec((_BN, d_in), lambda i: (i, 0)),
            pl.BlockSpec((_BN, d_in), lambda i: (i, 0)),
            pl.BlockSpec((_BN, 1), lambda i: (i, 0)),
            pl.BlockSpec((1, d_in), lambda i: (0, 0)),
            pl.BlockSpec(W.shape, lambda i: (0, 0)),
        ],
        out_specs=pl.BlockSpec((_BN, W.shape[1]), lambda i: (i, 0)),
        out_shape=jax.ShapeDtypeStruct((N, W.shape[1]), jnp.float32),
    )(agg, y, dis, b.reshape(1, -1), W)


def _tc_pool(agg, y, dis, b, batch2d):
    grid = (pl.cdiv(N, _BN),)
    d = y.shape[1]
    return pl.pallas_call(
        _pool_body,
        grid=grid,
        in_specs=[
            pl.BlockSpec((_BN, d), lambda i: (i, 0)),
            pl.BlockSpec((_BN, d), lambda i: (i, 0)),
            pl.BlockSpec((_BN, 1), lambda i: (i, 0)),
            pl.BlockSpec((1, d), lambda i: (0, 0)),
            pl.BlockSpec((_BN, 1), lambda i: (i, 0)),
        ],
        out_specs=[
            pl.BlockSpec((B, d), lambda i: (0, 0)),
            pl.BlockSpec((1, B), lambda i: (0, 0)),
        ],
        out_shape=[
            jax.ShapeDtypeStruct((B, d), jnp.float32),
            jax.ShapeDtypeStruct((1, B), jnp.float32),
        ],
    )(agg, y, dis, b.reshape(1, -1), batch2d)


def _tc_final(sums, cnt, Wp, bp):
    return pl.pallas_call(
        _final_body,
        out_shape=jax.ShapeDtypeStruct((B, Wp.shape[1]), jnp.float32),
    )(sums, cnt, Wp, bp.reshape(1, -1))


def _edge_aggregate(y, src, dst):
    return jnp.zeros_like(y).at[dst].add(y[src])


def kernel(x, edge_index, batch, W1, b1, W2, b2, W3, b3, Wp, bp):
    src = edge_index[0]
    dst = edge_index[1]
    deg = jnp.zeros((N,), jnp.float32).at[dst].add(1.0)

    y1, dis = _tc_first_layer(x, deg.reshape(N, 1), W1)
    agg1 = _edge_aggregate(y1, src, dst)
    y2 = _tc_mid_layer(agg1, y1, dis, b1, W2)
    agg2 = _edge_aggregate(y2, src, dst)
    y3 = _tc_mid_layer(agg2, y2, dis, b2, W3)
    agg3 = _edge_aggregate(y3, src, dst)
    sums, cnt = _tc_pool(agg3, y3, dis, b3, batch.reshape(N, 1))
    return _tc_final(sums, cnt, Wp, bp)


# speedup vs baseline: 1.6774x; 1.6774x over previous
import functools

import jax
import jax.numpy as jnp
from jax import lax
from jax.experimental import pallas as pl
from jax.experimental.pallas import tpu as pltpu
from jax.experimental.pallas import tpu_sc as plsc

N = 100000
Np = 98 * 1024
B = 512
_BN = 1024
_EB = 128
_PAD = 16 * _EB


def _first_layer_body(x_ref, deg_ref, w_ref, y_ref, dis_ref):
    dis = jax.lax.rsqrt(deg_ref[...] + 1.0)
    dis_ref[...] = dis
    y_ref[...] = dis * (x_ref[...] @ w_ref[...])


def _mid_layer_body(agg_ref, dis_ref, b_ref, w_ref, out_ref):
    dis = dis_ref[...]
    h = jnp.maximum(dis * agg_ref[...] + b_ref[...], 0.0)
    out_ref[...] = dis * (h @ w_ref[...])


def _pool_body(agg_ref, dis_ref, b_ref, batch_ref, sums_ref, cnt_ref):
    i = pl.program_id(0)

    @pl.when(i == 0)
    def _():
        sums_ref[...] = jnp.zeros_like(sums_ref)
        cnt_ref[...] = jnp.zeros_like(cnt_ref)

    dis = dis_ref[...]
    h = jnp.maximum(dis * agg_ref[...] + b_ref[...], 0.0)
    rows = i * _BN + jax.lax.broadcasted_iota(jnp.int32, (_BN, 1), 0)
    valid = rows < N
    seg = jax.lax.broadcasted_iota(jnp.int32, (_BN, B), 1)
    onehot = jnp.where((batch_ref[...] == seg) & valid, 1.0, 0.0)
    sums_ref[...] += jax.lax.dot_general(
        onehot, h, (((0,), (0,)), ((), ())), preferred_element_type=jnp.float32)
    cnt_ref[...] += jnp.sum(onehot, axis=0, keepdims=True)


def _final_body(sums_ref, cnt_ref, w_ref, b_ref, o_ref):
    pooled = sums_ref[...] / jnp.maximum(cnt_ref[...].T, 1.0)
    o_ref[...] = pooled @ w_ref[...] + b_ref[...]


def _tc_first_layer(x, deg, W):
    grid = (Np // _BN,)
    return pl.pallas_call(
        _first_layer_body,
        grid=grid,
        in_specs=[
            pl.BlockSpec((_BN, x.shape[1]), lambda i: (i, 0)),
            pl.BlockSpec((_BN, 1), lambda i: (i, 0)),
            pl.BlockSpec(W.shape, lambda i: (0, 0)),
        ],
        out_specs=[
            pl.BlockSpec((_BN, W.shape[1]), lambda i: (i, 0)),
            pl.BlockSpec((_BN, 1), lambda i: (i, 0)),
        ],
        out_shape=[
            jax.ShapeDtypeStruct((Np, W.shape[1]), jnp.float32),
            jax.ShapeDtypeStruct((Np, 1), jnp.float32),
        ],
    )(x, deg, W)


def _tc_mid_layer(agg, dis, b, W):
    grid = (Np // _BN,)
    d_in = agg.shape[1]
    return pl.pallas_call(
        _mid_layer_body,
        grid=grid,
        in_specs=[
            pl.BlockSpec((_BN, d_in), lambda i: (i, 0)),
            pl.BlockSpec((_BN, 1), lambda i: (i, 0)),
            pl.BlockSpec((1, d_in), lambda i: (0, 0)),
            pl.BlockSpec(W.shape, lambda i: (0, 0)),
        ],
        out_specs=pl.BlockSpec((_BN, W.shape[1]), lambda i: (i, 0)),
        out_shape=jax.ShapeDtypeStruct((Np, W.shape[1]), jnp.float32),
    )(agg, dis, b.reshape(1, -1), W)


def _tc_pool(agg, dis, b, batch2d):
    grid = (Np // _BN,)
    d = agg.shape[1]
    return pl.pallas_call(
        _pool_body,
        grid=grid,
        in_specs=[
            pl.BlockSpec((_BN, d), lambda i: (i, 0)),
            pl.BlockSpec((_BN, 1), lambda i: (i, 0)),
            pl.BlockSpec((1, d), lambda i: (0, 0)),
            pl.BlockSpec((_BN, 1), lambda i: (i, 0)),
        ],
        out_specs=[
            pl.BlockSpec((B, d), lambda i: (0, 0)),
            pl.BlockSpec((1, B), lambda i: (0, 0)),
        ],
        out_shape=[
            jax.ShapeDtypeStruct((B, d), jnp.float32),
            jax.ShapeDtypeStruct((1, B), jnp.float32),
        ],
    )(agg, dis, b.reshape(1, -1), batch2d)


def _tc_final(sums, cnt, Wp, bp):
    return pl.pallas_call(
        _final_body,
        out_shape=jax.ShapeDtypeStruct((B, Wp.shape[1]), jnp.float32),
    )(sums, cnt, Wp, bp.reshape(1, -1))


def _chunk_rows(d):
    return {128: 448, 256: 224}[d]


def _sc_edge_aggregate(y, src_p, dst_p, offsets):
    d = y.shape[1]
    Rt = _chunk_rows(d)
    nsub = Np // Rt
    rounds = nsub // 32
    mesh = plsc.VectorSubcoreMesh(core_axis_name="c", subcore_axis_name="s")

    @functools.partial(
        pl.kernel,
        out_type=jax.ShapeDtypeStruct((Np, d), jnp.float32),
        mesh=mesh,
        scratch_types=[
            pltpu.VMEM((Rt + 1, d), jnp.float32),
            pltpu.VMEM((_EB, d), jnp.float32),
            pltpu.VMEM((_EB,), jnp.int32),
            pltpu.VMEM((_EB,), jnp.int32),
            pltpu.VMEM((512,), jnp.int32),
            pltpu.SemaphoreType.DMA,
        ],
    )
    def k(y_hbm, src_hbm, dst_hbm, off_hbm, out_hbm,
          acc, rows, src_idx, dst_raw, offs, sem):
        cid = lax.axis_index("c")
        tid = lax.axis_index("s")
        wid = cid * 16 + tid
        pltpu.sync_copy(off_hbm, offs)

        def round_body(r, carry):
            g = r * 32 + wid
            base = g * Rt
            pltpu.sync_copy(y_hbm.at[pl.ds(base, Rt)], acc.at[pl.ds(0, Rt)])
            ov = offs[pl.ds(g, 16)]
            e_lo = ov[0]
            e_hi = ov[1]
            e0 = (e_lo // _EB) * _EB
            nblk = (e_hi - e0 + _EB - 1) // _EB

            def blk(j, carry2):
                e = e0 + j * _EB
                pltpu.sync_copy(src_hbm.at[pl.ds(e, _EB)], src_idx)
                pltpu.sync_copy(dst_hbm.at[pl.ds(e, _EB)], dst_raw)
                pltpu.async_copy(y_hbm.at[src_idx], rows, sem).wait()

                def grp(kk, carry3):
                    dv = dst_raw[pl.ds(kk * 16, 16)]
                    in_r = (dv >= base) & (dv < base + Rt)
                    dloc16 = jnp.where(in_r, dv - base, Rt)
                    for l in range(16):
                        dl = dloc16[l]
                        rk = kk * 16 + l
                        for s in range(d // 16):
                            plsc.addupdate(
                                acc.at[dl, pl.ds(s * 16, 16)],
                                rows[rk, pl.ds(s * 16, 16)])
                    return carry3
                lax.fori_loop(0, _EB // 16, grp, 0)
                return carry2
            lax.fori_loop(0, nblk, blk, 0)
            pltpu.sync_copy(acc.at[pl.ds(0, Rt)], out_hbm.at[pl.ds(base, Rt)])
            return carry
        lax.fori_loop(0, rounds, round_body, 0)

    return k(y, src_p, dst_p, offsets)


def kernel(x, edge_index, batch, W1, b1, W2, b2, W3, b3, Wp, bp):
    src = edge_index[0]
    dst = edge_index[1]
    deg = jnp.zeros((N,), jnp.float32).at[dst].add(1.0)

    x_p = jnp.concatenate([x, jnp.zeros((Np - N, x.shape[1]), jnp.float32)])
    deg_p = jnp.concatenate([deg, jnp.zeros((Np - N,), jnp.float32)])
    batch_p = jnp.concatenate([batch, jnp.zeros((Np - N,), jnp.int32)])

    perm = jnp.argsort(dst)
    src_s = src[perm]
    dst_s = dst[perm]
    src_p = jnp.concatenate([src_s, jnp.zeros((_PAD,), jnp.int32)])
    dst_p = jnp.concatenate([dst_s, jnp.full((_PAD,), N, jnp.int32)])
    bounds = jnp.arange(512, dtype=jnp.int32)
    off1 = jnp.searchsorted(dst_s, bounds * _chunk_rows(128)).astype(jnp.int32)
    off2 = jnp.searchsorted(dst_s, bounds * _chunk_rows(256)).astype(jnp.int32)

    y1, dis = _tc_first_layer(x_p, deg_p.reshape(Np, 1), W1)
    agg1 = _sc_edge_aggregate(y1, src_p, dst_p, off1)
    y2 = _tc_mid_layer(agg1, dis, b1, W2)
    agg2 = _sc_edge_aggregate(y2, src_p, dst_p, off2)
    y3 = _tc_mid_layer(agg2, dis, b2, W3)
    agg3 = _sc_edge_aggregate(y3, src_p, dst_p, off2)
    sums, cnt = _tc_pool(agg3, dis, b3, batch_p.reshape(Np, 1))
    return _tc_final(sums, cnt, Wp, bp)


# speedup vs baseline: 1.9102x; 1.1388x over previous
import functools

import jax
import jax.numpy as jnp
from jax import lax
from jax.experimental import pallas as pl
from jax.experimental.pallas import tpu as pltpu
from jax.experimental.pallas import tpu_sc as plsc

N = 100000
Np = 98 * 1024
B = 512
_BN = 1024
_EB = 128
_PAD = 16 * _EB


def _first_layer_body(x_ref, deg_ref, w_ref, y_ref, dis_ref):
    dis = jax.lax.rsqrt(deg_ref[...] + 1.0)
    dis_ref[...] = dis
    y_ref[...] = dis * (x_ref[...] @ w_ref[...])


def _mid_layer_body(agg_ref, dis_ref, b_ref, w_ref, out_ref):
    dis = dis_ref[...]
    h = jnp.maximum(dis * agg_ref[...] + b_ref[...], 0.0)
    out_ref[...] = dis * (h @ w_ref[...])


def _pool_body(agg_ref, dis_ref, b_ref, batch_ref, sums_ref, cnt_ref):
    i = pl.program_id(0)

    @pl.when(i == 0)
    def _():
        sums_ref[...] = jnp.zeros_like(sums_ref)
        cnt_ref[...] = jnp.zeros_like(cnt_ref)

    dis = dis_ref[...]
    h = jnp.maximum(dis * agg_ref[...] + b_ref[...], 0.0)
    rows = i * _BN + jax.lax.broadcasted_iota(jnp.int32, (_BN, 1), 0)
    valid = rows < N
    seg = jax.lax.broadcasted_iota(jnp.int32, (_BN, B), 1)
    onehot = jnp.where((batch_ref[...] == seg) & valid, 1.0, 0.0)
    sums_ref[...] += jax.lax.dot_general(
        onehot, h, (((0,), (0,)), ((), ())), preferred_element_type=jnp.float32)
    cnt_ref[...] += jnp.sum(onehot, axis=0, keepdims=True)


def _final_body(sums_ref, cnt_ref, w_ref, b_ref, o_ref):
    pooled = sums_ref[...] / jnp.maximum(cnt_ref[...].T, 1.0)
    o_ref[...] = pooled @ w_ref[...] + b_ref[...]


def _tc_first_layer(x, deg, W):
    grid = (Np // _BN,)
    return pl.pallas_call(
        _first_layer_body,
        grid=grid,
        in_specs=[
            pl.BlockSpec((_BN, x.shape[1]), lambda i: (i, 0)),
            pl.BlockSpec((_BN, 1), lambda i: (i, 0)),
            pl.BlockSpec(W.shape, lambda i: (0, 0)),
        ],
        out_specs=[
            pl.BlockSpec((_BN, W.shape[1]), lambda i: (i, 0)),
            pl.BlockSpec((_BN, 1), lambda i: (i, 0)),
        ],
        out_shape=[
            jax.ShapeDtypeStruct((Np, W.shape[1]), jnp.float32),
            jax.ShapeDtypeStruct((Np, 1), jnp.float32),
        ],
    )(x, deg, W)


def _tc_mid_layer(agg, dis, b, W):
    grid = (Np // _BN,)
    d_in = agg.shape[1]
    return pl.pallas_call(
        _mid_layer_body,
        grid=grid,
        in_specs=[
            pl.BlockSpec((_BN, d_in), lambda i: (i, 0)),
            pl.BlockSpec((_BN, 1), lambda i: (i, 0)),
            pl.BlockSpec((1, d_in), lambda i: (0, 0)),
            pl.BlockSpec(W.shape, lambda i: (0, 0)),
        ],
        out_specs=pl.BlockSpec((_BN, W.shape[1]), lambda i: (i, 0)),
        out_shape=jax.ShapeDtypeStruct((Np, W.shape[1]), jnp.float32),
    )(agg, dis, b.reshape(1, -1), W)


def _tc_pool(agg, dis, b, batch2d):
    grid = (Np // _BN,)
    d = agg.shape[1]
    return pl.pallas_call(
        _pool_body,
        grid=grid,
        in_specs=[
            pl.BlockSpec((_BN, d), lambda i: (i, 0)),
            pl.BlockSpec((_BN, 1), lambda i: (i, 0)),
            pl.BlockSpec((1, d), lambda i: (0, 0)),
            pl.BlockSpec((_BN, 1), lambda i: (i, 0)),
        ],
        out_specs=[
            pl.BlockSpec((B, d), lambda i: (0, 0)),
            pl.BlockSpec((1, B), lambda i: (0, 0)),
        ],
        out_shape=[
            jax.ShapeDtypeStruct((B, d), jnp.float32),
            jax.ShapeDtypeStruct((1, B), jnp.float32),
        ],
    )(agg, dis, b.reshape(1, -1), batch2d)


def _tc_final(sums, cnt, Wp, bp):
    return pl.pallas_call(
        _final_body,
        out_shape=jax.ShapeDtypeStruct((B, Wp.shape[1]), jnp.float32),
    )(sums, cnt, Wp, bp.reshape(1, -1))


def _chunk_rows(d):
    return {128: 448, 256: 224}[d]


def _sc_edge_aggregate(y, src_p, dst_p, offsets):
    d = y.shape[1]
    Rt = _chunk_rows(d)
    nsub = Np // Rt
    rounds = nsub // 32
    mesh = plsc.VectorSubcoreMesh(core_axis_name="c", subcore_axis_name="s")

    @functools.partial(
        pl.kernel,
        out_type=jax.ShapeDtypeStruct((Np, d), jnp.float32),
        mesh=mesh,
        scratch_types=[
            pltpu.VMEM((Rt + 1, d), jnp.float32),
            pltpu.VMEM((_EB, d), jnp.float32),
            pltpu.VMEM((_EB, d), jnp.float32),
            pltpu.VMEM((_EB,), jnp.int32),
            pltpu.VMEM((_EB,), jnp.int32),
            pltpu.VMEM((_EB,), jnp.int32),
            pltpu.VMEM((_EB,), jnp.int32),
            pltpu.VMEM((512,), jnp.int32),
            pltpu.SemaphoreType.DMA,
            pltpu.SemaphoreType.DMA,
        ],
    )
    def k(y_hbm, src_hbm, dst_hbm, off_hbm, out_hbm,
          acc, rows0, rows1, si0, si1, dr0, dr1, offs, sem0, sem1):
        cid = lax.axis_index("c")
        tid = lax.axis_index("s")
        wid = cid * 16 + tid
        pltpu.sync_copy(off_hbm, offs)
        rows_b = (rows0, rows1)
        si_b = (si0, si1)
        dr_b = (dr0, dr1)
        sem_b = (sem0, sem1)

        def round_body(r, carry):
            g = r * 32 + wid
            base = g * Rt
            pltpu.sync_copy(y_hbm.at[pl.ds(base, Rt)], acc.at[pl.ds(0, Rt)])
            ov = offs[pl.ds(g, 16)]
            e_lo = ov[0]
            e_hi = ov[1]
            e0 = (e_lo // _EB) * _EB
            nblk = (e_hi - e0 + _EB - 1) // _EB

            def issue(j, b):
                e = e0 + j * _EB
                pltpu.sync_copy(src_hbm.at[pl.ds(e, _EB)], si_b[b])
                pltpu.sync_copy(dst_hbm.at[pl.ds(e, _EB)], dr_b[b])
                pltpu.async_copy(y_hbm.at[si_b[b]], rows_b[b], sem_b[b])

            def accumulate(b):
                rows = rows_b[b]
                dst_raw = dr_b[b]

                def grp(kk, carry3):
                    dv = dst_raw[pl.ds(kk * 16, 16)]
                    in_r = (dv >= base) & (dv < base + Rt)
                    dloc16 = jnp.where(in_r, dv - base, Rt)
                    for l in range(16):
                        dl = dloc16[l]
                        rk = kk * 16 + l
                        for s in range(d // 16):
                            plsc.addupdate(
                                acc.at[dl, pl.ds(s * 16, 16)],
                                rows[rk, pl.ds(s * 16, 16)])
                    return carry3
                lax.fori_loop(0, _EB // 16, grp, 0)

            @pl.when(nblk > 0)
            def _():
                issue(0, 0)

            @pl.when(nblk > 1)
            def _():
                issue(1, 1)

            def pair(p, carry2):
                for b in range(2):
                    j = p * 2 + b

                    @pl.when(j < nblk)
                    def _(b=b, j=j):
                        pltpu.make_async_copy(
                            y_hbm.at[pl.ds(0, _EB)], rows_b[b],
                            sem_b[b]).wait()
                        accumulate(b)

                        @pl.when(j + 2 < nblk)
                        def _():
                            issue(j + 2, b)
                return carry2
            lax.fori_loop(0, (nblk + 1) // 2, pair, 0)
            pltpu.sync_copy(acc.at[pl.ds(0, Rt)], out_hbm.at[pl.ds(base, Rt)])
            return carry
        lax.fori_loop(0, rounds, round_body, 0)

    return k(y, src_p, dst_p, offsets)


def kernel(x, edge_index, batch, W1, b1, W2, b2, W3, b3, Wp, bp):
    src = edge_index[0]
    dst = edge_index[1]
    deg = jnp.zeros((N,), jnp.float32).at[dst].add(1.0)

    x_p = jnp.concatenate([x, jnp.zeros((Np - N, x.shape[1]), jnp.float32)])
    deg_p = jnp.concatenate([deg, jnp.zeros((Np - N,), jnp.float32)])
    batch_p = jnp.concatenate([batch, jnp.zeros((Np - N,), jnp.int32)])

    perm = jnp.argsort(dst)
    src_s = src[perm]
    dst_s = dst[perm]
    src_p = jnp.concatenate([src_s, jnp.zeros((_PAD,), jnp.int32)])
    dst_p = jnp.concatenate([dst_s, jnp.full((_PAD,), N, jnp.int32)])
    bounds = jnp.arange(512, dtype=jnp.int32)
    off1 = jnp.searchsorted(dst_s, bounds * _chunk_rows(128)).astype(jnp.int32)
    off2 = jnp.searchsorted(dst_s, bounds * _chunk_rows(256)).astype(jnp.int32)

    y1, dis = _tc_first_layer(x_p, deg_p.reshape(Np, 1), W1)
    agg1 = _sc_edge_aggregate(y1, src_p, dst_p, off1)
    y2 = _tc_mid_layer(agg1, dis, b1, W2)
    agg2 = _sc_edge_aggregate(y2, src_p, dst_p, off2)
    y3 = _tc_mid_layer(agg2, dis, b2, W3)
    agg3 = _sc_edge_aggregate(y3, src_p, dst_p, off2)
    sums, cnt = _tc_pool(agg3, dis, b3, batch_p.reshape(Np, 1))
    return _tc_final(sums, cnt, Wp, bp)


# speedup vs baseline: 1.9805x; 1.0368x over previous
import functools

import jax
import jax.numpy as jnp
from jax import lax
from jax.experimental import pallas as pl
from jax.experimental.pallas import tpu as pltpu
from jax.experimental.pallas import tpu_sc as plsc

N = 100000
Np = 98 * 1024
B = 512
_BN = 1024
_EB = 128
_NG = 8
_PAD = 16 * _EB


def _first_layer_body(x_ref, deg_ref, w_ref, y_ref, dis_ref):
    dis = jax.lax.rsqrt(deg_ref[...] + 1.0)
    dis_ref[...] = dis
    y_ref[...] = dis * (x_ref[...] @ w_ref[...])


def _mid_layer_body(agg_ref, dis_ref, b_ref, w_ref, out_ref):
    dis = dis_ref[...]
    h = jnp.maximum(dis * agg_ref[...] + b_ref[...], 0.0)
    out_ref[...] = dis * (h @ w_ref[...])


def _pool_body(agg_ref, dis_ref, b_ref, batch_ref, sums_ref, cnt_ref):
    i = pl.program_id(0)

    @pl.when(i == 0)
    def _():
        sums_ref[...] = jnp.zeros_like(sums_ref)
        cnt_ref[...] = jnp.zeros_like(cnt_ref)

    dis = dis_ref[...]
    h = jnp.maximum(dis * agg_ref[...] + b_ref[...], 0.0)
    rows = i * _BN + jax.lax.broadcasted_iota(jnp.int32, (_BN, 1), 0)
    valid = rows < N
    seg = jax.lax.broadcasted_iota(jnp.int32, (_BN, B), 1)
    onehot = jnp.where((batch_ref[...] == seg) & valid, 1.0, 0.0)
    sums_ref[...] += jax.lax.dot_general(
        onehot, h, (((0,), (0,)), ((), ())), preferred_element_type=jnp.float32)
    cnt_ref[...] += jnp.sum(onehot, axis=0, keepdims=True)


def _final_body(sums_ref, cnt_ref, w_ref, b_ref, o_ref):
    pooled = sums_ref[...] / jnp.maximum(cnt_ref[...].T, 1.0)
    o_ref[...] = pooled @ w_ref[...] + b_ref[...]


def _tc_first_layer(x, deg, W):
    grid = (Np // _BN,)
    return pl.pallas_call(
        _first_layer_body,
        grid=grid,
        in_specs=[
            pl.BlockSpec((_BN, x.shape[1]), lambda i: (i, 0)),
            pl.BlockSpec((_BN, 1), lambda i: (i, 0)),
            pl.BlockSpec(W.shape, lambda i: (0, 0)),
        ],
        out_specs=[
            pl.BlockSpec((_BN, W.shape[1]), lambda i: (i, 0)),
            pl.BlockSpec((_BN, 1), lambda i: (i, 0)),
        ],
        out_shape=[
            jax.ShapeDtypeStruct((Np, W.shape[1]), jnp.float32),
            jax.ShapeDtypeStruct((Np, 1), jnp.float32),
        ],
    )(x, deg, W)


def _tc_mid_layer(agg, dis, b, W):
    grid = (Np // _BN,)
    d_in = agg.shape[1]
    return pl.pallas_call(
        _mid_layer_body,
        grid=grid,
        in_specs=[
            pl.BlockSpec((_BN, d_in), lambda i: (i, 0)),
            pl.BlockSpec((_BN, 1), lambda i: (i, 0)),
            pl.BlockSpec((1, d_in), lambda i: (0, 0)),
            pl.BlockSpec(W.shape, lambda i: (0, 0)),
        ],
        out_specs=pl.BlockSpec((_BN, W.shape[1]), lambda i: (i, 0)),
        out_shape=jax.ShapeDtypeStruct((Np, W.shape[1]), jnp.float32),
    )(agg, dis, b.reshape(1, -1), W)


def _tc_pool(agg, dis, b, batch2d):
    grid = (Np // _BN,)
    d = agg.shape[1]
    return pl.pallas_call(
        _pool_body,
        grid=grid,
        in_specs=[
            pl.BlockSpec((_BN, d), lambda i: (i, 0)),
            pl.BlockSpec((_BN, 1), lambda i: (i, 0)),
            pl.BlockSpec((1, d), lambda i: (0, 0)),
            pl.BlockSpec((_BN, 1), lambda i: (i, 0)),
        ],
        out_specs=[
            pl.BlockSpec((B, d), lambda i: (0, 0)),
            pl.BlockSpec((1, B), lambda i: (0, 0)),
        ],
        out_shape=[
            jax.ShapeDtypeStruct((B, d), jnp.float32),
            jax.ShapeDtypeStruct((1, B), jnp.float32),
        ],
    )(agg, dis, b.reshape(1, -1), batch2d)


def _tc_final(sums, cnt, Wp, bp):
    return pl.pallas_call(
        _final_body,
        out_shape=jax.ShapeDtypeStruct((B, Wp.shape[1]), jnp.float32),
    )(sums, cnt, Wp, bp.reshape(1, -1))


def _chunk_rows(d):
    return {128: 448, 256: 224}[d]


def _sc_edge_aggregate(y, src_p, dst_p, offsets):
    d = y.shape[1]
    Rt = _chunk_rows(d)
    nsub = Np // Rt
    rounds = nsub // 32
    mesh = plsc.VectorSubcoreMesh(core_axis_name="c", subcore_axis_name="s")

    @functools.partial(
        pl.kernel,
        out_type=jax.ShapeDtypeStruct((Np, d), jnp.float32),
        mesh=mesh,
        scratch_types=[
            pltpu.VMEM((Rt + 1, d), jnp.float32),
            pltpu.VMEM((_EB, d), jnp.float32),
            pltpu.VMEM((_EB, d), jnp.float32),
            pltpu.VMEM((_NG * _EB,), jnp.int32),
            pltpu.VMEM((_NG * _EB,), jnp.int32),
            pltpu.VMEM((512,), jnp.int32),
            pltpu.SemaphoreType.DMA,
            pltpu.SemaphoreType.DMA,
        ],
    )
    def k(y_hbm, src_hbm, dst_hbm, off_hbm, out_hbm,
          acc, rows0, rows1, si_g, dr_g, offs, sem0, sem1):
        cid = lax.axis_index("c")
        tid = lax.axis_index("s")
        wid = cid * 16 + tid
        pltpu.sync_copy(off_hbm, offs)
        rows_b = (rows0, rows1)
        sem_b = (sem0, sem1)

        def round_body(r, carry):
            g = r * 32 + wid
            base = g * Rt
            pltpu.sync_copy(y_hbm.at[pl.ds(base, Rt)], acc.at[pl.ds(0, Rt)])
            ov = offs[pl.ds(g, 16)]
            e_lo = ov[0]
            e_hi = ov[1]
            e0 = (e_lo // _EB) * _EB
            nblk = (e_hi - e0 + _EB - 1) // _EB

            def issue(lb, slot):
                pltpu.async_copy(
                    y_hbm.at[si_g.at[pl.ds(lb * _EB, _EB)]],
                    rows_b[slot], sem_b[slot])

            def accumulate(lb, slot):
                rows = rows_b[slot]

                def grp(kk, carry3):
                    dv = dr_g[pl.ds(lb * _EB + kk * 16, 16)]
                    in_r = (dv >= base) & (dv < base + Rt)
                    dloc16 = jnp.where(in_r, dv - base, Rt)
                    for l in range(16):
                        dl = dloc16[l]
                        rk = kk * 16 + l
                        for s in range(d // 16):
                            plsc.addupdate(
                                acc.at[dl, pl.ds(s * 16, 16)],
                                rows[rk, pl.ds(s * 16, 16)])
                    return carry3
                lax.fori_loop(0, _EB // 16, grp, 0)

            def group_body(gi, carry2):
                jbase = gi * _NG
                e = e0 + jbase * _EB
                pltpu.sync_copy(src_hbm.at[pl.ds(e, _NG * _EB)], si_g)
                pltpu.sync_copy(dst_hbm.at[pl.ds(e, _NG * _EB)], dr_g)

                @pl.when(jbase + 0 < nblk)
                def _():
                    issue(0, 0)

                @pl.when(jbase + 1 < nblk)
                def _():
                    issue(1, 1)

                for lb in range(_NG):
                    slot = lb % 2

                    @pl.when(jbase + lb < nblk)
                    def _(lb=lb, slot=slot):
                        pltpu.make_async_copy(
                            y_hbm.at[pl.ds(0, _EB)], rows_b[slot],
                            sem_b[slot]).wait()
                        accumulate(lb, slot)
                        if lb + 2 < _NG:
                            @pl.when(jbase + lb + 2 < nblk)
                            def _():
                                issue(lb + 2, slot)
                return carry2
            lax.fori_loop(0, (nblk + _NG - 1) // _NG, group_body, 0)
            pltpu.sync_copy(acc.at[pl.ds(0, Rt)], out_hbm.at[pl.ds(base, Rt)])
            return carry
        lax.fori_loop(0, rounds, round_body, 0)

    return k(y, src_p, dst_p, offsets)


def kernel(x, edge_index, batch, W1, b1, W2, b2, W3, b3, Wp, bp):
    src = edge_index[0]
    dst = edge_index[1]
    deg = jnp.zeros((N,), jnp.float32).at[dst].add(1.0)

    x_p = jnp.concatenate([x, jnp.zeros((Np - N, x.shape[1]), jnp.float32)])
    deg_p = jnp.concatenate([deg, jnp.zeros((Np - N,), jnp.float32)])
    batch_p = jnp.concatenate([batch, jnp.zeros((Np - N,), jnp.int32)])

    perm = jnp.argsort(dst)
    src_s = src[perm]
    dst_s = dst[perm]
    src_p = jnp.concatenate([src_s, jnp.zeros((_PAD,), jnp.int32)])
    dst_p = jnp.concatenate([dst_s, jnp.full((_PAD,), N, jnp.int32)])
    bounds = jnp.arange(512, dtype=jnp.int32)
    off1 = jnp.searchsorted(dst_s, bounds * _chunk_rows(128)).astype(jnp.int32)
    off2 = jnp.searchsorted(dst_s, bounds * _chunk_rows(256)).astype(jnp.int32)

    y1, dis = _tc_first_layer(x_p, deg_p.reshape(Np, 1), W1)
    agg1 = _sc_edge_aggregate(y1, src_p, dst_p, off1)
    y2 = _tc_mid_layer(agg1, dis, b1, W2)
    agg2 = _sc_edge_aggregate(y2, src_p, dst_p, off2)
    y3 = _tc_mid_layer(agg2, dis, b2, W3)
    agg3 = _sc_edge_aggregate(y3, src_p, dst_p, off2)
    sums, cnt = _tc_pool(agg3, dis, b3, batch_p.reshape(Np, 1))
    return _tc_final(sums, cnt, Wp, bp)


# speedup vs baseline: 1.9997x; 1.0097x over previous
import functools

import jax
import jax.numpy as jnp
from jax import lax
from jax.experimental import pallas as pl
from jax.experimental.pallas import tpu as pltpu
from jax.experimental.pallas import tpu_sc as plsc

N = 100000
Np = 98 * 1024
B = 512
_BN = 1024
_EB = 128
_NG = 8
_PAD = 16 * _EB


def _first_layer_body(x_ref, deg_ref, w_ref, y_ref, dis_ref):
    dis = jax.lax.rsqrt(deg_ref[...] + 1.0)
    dis_ref[...] = dis
    y_ref[...] = dis * (x_ref[...] @ w_ref[...])


def _mid_layer_body(agg_ref, dis_ref, b_ref, w_ref, out_ref):
    dis = dis_ref[...]
    h = jnp.maximum(dis * agg_ref[...] + b_ref[...], 0.0)
    out_ref[...] = dis * (h @ w_ref[...])


def _pool_body(agg_ref, dis_ref, b_ref, batch_ref, sums_ref, cnt_ref):
    i = pl.program_id(0)

    @pl.when(i == 0)
    def _():
        sums_ref[...] = jnp.zeros_like(sums_ref)
        cnt_ref[...] = jnp.zeros_like(cnt_ref)

    dis = dis_ref[...]
    h = jnp.maximum(dis * agg_ref[...] + b_ref[...], 0.0)
    rows = i * _BN + jax.lax.broadcasted_iota(jnp.int32, (_BN, 1), 0)
    valid = rows < N
    seg = jax.lax.broadcasted_iota(jnp.int32, (_BN, B), 1)
    onehot = jnp.where((batch_ref[...] == seg) & valid, 1.0, 0.0)
    sums_ref[...] += jax.lax.dot_general(
        onehot, h, (((0,), (0,)), ((), ())), preferred_element_type=jnp.float32)
    cnt_ref[...] += jnp.sum(onehot, axis=0, keepdims=True)


def _final_body(sums_ref, cnt_ref, w_ref, b_ref, o_ref):
    pooled = sums_ref[...] / jnp.maximum(cnt_ref[...].T, 1.0)
    o_ref[...] = pooled @ w_ref[...] + b_ref[...]


def _tc_first_layer(x, deg, W):
    grid = (Np // _BN,)
    return pl.pallas_call(
        _first_layer_body,
        grid=grid,
        in_specs=[
            pl.BlockSpec((_BN, x.shape[1]), lambda i: (i, 0)),
            pl.BlockSpec((_BN, 1), lambda i: (i, 0)),
            pl.BlockSpec(W.shape, lambda i: (0, 0)),
        ],
        out_specs=[
            pl.BlockSpec((_BN, W.shape[1]), lambda i: (i, 0)),
            pl.BlockSpec((_BN, 1), lambda i: (i, 0)),
        ],
        out_shape=[
            jax.ShapeDtypeStruct((Np, W.shape[1]), jnp.float32),
            jax.ShapeDtypeStruct((Np, 1), jnp.float32),
        ],
    )(x, deg, W)


def _tc_mid_layer(agg, dis, b, W):
    grid = (Np // _BN,)
    d_in = agg.shape[1]
    return pl.pallas_call(
        _mid_layer_body,
        grid=grid,
        in_specs=[
            pl.BlockSpec((_BN, d_in), lambda i: (i, 0)),
            pl.BlockSpec((_BN, 1), lambda i: (i, 0)),
            pl.BlockSpec((1, d_in), lambda i: (0, 0)),
            pl.BlockSpec(W.shape, lambda i: (0, 0)),
        ],
        out_specs=pl.BlockSpec((_BN, W.shape[1]), lambda i: (i, 0)),
        out_shape=jax.ShapeDtypeStruct((Np, W.shape[1]), jnp.float32),
    )(agg, dis, b.reshape(1, -1), W)


def _tc_pool(agg, dis, b, batch2d):
    grid = (Np // _BN,)
    d = agg.shape[1]
    return pl.pallas_call(
        _pool_body,
        grid=grid,
        in_specs=[
            pl.BlockSpec((_BN, d), lambda i: (i, 0)),
            pl.BlockSpec((_BN, 1), lambda i: (i, 0)),
            pl.BlockSpec((1, d), lambda i: (0, 0)),
            pl.BlockSpec((_BN, 1), lambda i: (i, 0)),
        ],
        out_specs=[
            pl.BlockSpec((B, d), lambda i: (0, 0)),
            pl.BlockSpec((1, B), lambda i: (0, 0)),
        ],
        out_shape=[
            jax.ShapeDtypeStruct((B, d), jnp.float32),
            jax.ShapeDtypeStruct((1, B), jnp.float32),
        ],
    )(agg, dis, b.reshape(1, -1), batch2d)


def _tc_final(sums, cnt, Wp, bp):
    return pl.pallas_call(
        _final_body,
        out_shape=jax.ShapeDtypeStruct((B, Wp.shape[1]), jnp.float32),
    )(sums, cnt, Wp, bp.reshape(1, -1))


def _chunk_rows(d):
    return {128: 448, 256: 224}[d]


def _sc_edge_aggregate(y, src_p, dst_p, offsets):
    d = y.shape[1]
    Rt = _chunk_rows(d)
    eb = {128: 2 * _EB, 256: _EB}[d]
    nsub = Np // Rt
    rounds = nsub // 32
    mesh = plsc.VectorSubcoreMesh(core_axis_name="c", subcore_axis_name="s")

    @functools.partial(
        pl.kernel,
        out_type=jax.ShapeDtypeStruct((Np, d), jnp.float32),
        mesh=mesh,
        scratch_types=[
            pltpu.VMEM((Rt + 1, d), jnp.float32),
            pltpu.VMEM((eb, d), jnp.float32),
            pltpu.VMEM((eb, d), jnp.float32),
            pltpu.VMEM((_NG * eb,), jnp.int32),
            pltpu.VMEM((_NG * eb,), jnp.int32),
            pltpu.VMEM((512,), jnp.int32),
            pltpu.SemaphoreType.DMA,
            pltpu.SemaphoreType.DMA,
        ],
    )
    def k(y_hbm, src_hbm, dst_hbm, off_hbm, out_hbm,
          acc, rows0, rows1, si_g, dr_g, offs, sem0, sem1):
        cid = lax.axis_index("c")
        tid = lax.axis_index("s")
        wid = cid * 16 + tid
        pltpu.sync_copy(off_hbm, offs)
        rows_b = (rows0, rows1)
        sem_b = (sem0, sem1)

        def round_body(r, carry):
            g = r * 32 + wid
            base = g * Rt
            pltpu.sync_copy(y_hbm.at[pl.ds(base, Rt)], acc.at[pl.ds(0, Rt)])
            ov = offs[pl.ds(g, 16)]
            e_lo = ov[0]
            e_hi = ov[1]
            e0 = (e_lo // eb) * eb
            nblk = (e_hi - e0 + eb - 1) // eb

            def issue(lb, slot):
                pltpu.async_copy(
                    y_hbm.at[si_g.at[pl.ds(lb * eb, eb)]],
                    rows_b[slot], sem_b[slot])

            def accumulate(lb, slot):
                rows = rows_b[slot]

                def grp(kk, carry3):
                    dv = dr_g[pl.ds(lb * eb + kk * 16, 16)]
                    in_r = (dv >= base) & (dv < base + Rt)
                    dloc16 = jnp.where(in_r, dv - base, Rt)
                    for l in range(16):
                        dl = dloc16[l]
                        rk = kk * 16 + l
                        for s in range(d // 16):
                            plsc.addupdate(
                                acc.at[dl, pl.ds(s * 16, 16)],
                                rows[rk, pl.ds(s * 16, 16)])
                    return carry3
                lax.fori_loop(0, eb // 16, grp, 0)

            def group_body(gi, carry2):
                jbase = gi * _NG
                e = e0 + jbase * eb
                pltpu.sync_copy(src_hbm.at[pl.ds(e, _NG * eb)], si_g)
                pltpu.sync_copy(dst_hbm.at[pl.ds(e, _NG * eb)], dr_g)

                @pl.when(jbase + 0 < nblk)
                def _():
                    issue(0, 0)

                @pl.when(jbase + 1 < nblk)
                def _():
                    issue(1, 1)

                for lb in range(_NG):
                    slot = lb % 2

                    @pl.when(jbase + lb < nblk)
                    def _(lb=lb, slot=slot):
                        pltpu.make_async_copy(
                            y_hbm.at[pl.ds(0, eb)], rows_b[slot],
                            sem_b[slot]).wait()
                        accumulate(lb, slot)
                        if lb + 2 < _NG:
                            @pl.when(jbase + lb + 2 < nblk)
                            def _():
                                issue(lb + 2, slot)
                return carry2
            lax.fori_loop(0, (nblk + _NG - 1) // _NG, group_body, 0)
            pltpu.sync_copy(acc.at[pl.ds(0, Rt)], out_hbm.at[pl.ds(base, Rt)])
            return carry
        lax.fori_loop(0, rounds, round_body, 0)

    return k(y, src_p, dst_p, offsets)


def kernel(x, edge_index, batch, W1, b1, W2, b2, W3, b3, Wp, bp):
    src = edge_index[0]
    dst = edge_index[1]
    deg = jnp.zeros((N,), jnp.float32).at[dst].add(1.0)

    x_p = jnp.concatenate([x, jnp.zeros((Np - N, x.shape[1]), jnp.float32)])
    deg_p = jnp.concatenate([deg, jnp.zeros((Np - N,), jnp.float32)])
    batch_p = jnp.concatenate([batch, jnp.zeros((Np - N,), jnp.int32)])

    dst_s, src_s = lax.sort((dst, src), num_keys=1)
    src_p = jnp.concatenate([src_s, jnp.zeros((_PAD,), jnp.int32)])
    dst_p = jnp.concatenate([dst_s, jnp.full((_PAD,), N, jnp.int32)])
    bounds = jnp.arange(512, dtype=jnp.int32)
    off1 = jnp.searchsorted(dst_s, bounds * _chunk_rows(128)).astype(jnp.int32)
    off2 = jnp.searchsorted(dst_s, bounds * _chunk_rows(256)).astype(jnp.int32)

    y1, dis = _tc_first_layer(x_p, deg_p.reshape(Np, 1), W1)
    agg1 = _sc_edge_aggregate(y1, src_p, dst_p, off1)
    y2 = _tc_mid_layer(agg1, dis, b1, W2)
    agg2 = _sc_edge_aggregate(y2, src_p, dst_p, off2)
    y3 = _tc_mid_layer(agg2, dis, b2, W3)
    agg3 = _sc_edge_aggregate(y3, src_p, dst_p, off2)
    sums, cnt = _tc_pool(agg3, dis, b3, batch_p.reshape(Np, 1))
    return _tc_final(sums, cnt, Wp, bp)
